# baseline (speedup 1.0000x reference)
import functools

import jax
import jax.numpy as jnp
from jax import lax
from jax.experimental import pallas as pl
from jax.experimental.pallas import tpu as pltpu
from jax.experimental.pallas import tpu_sc as plsc

_NUM_BINS = 1000000
_HASH_MULT = 2654435761
_EMBED_DIM = 32
_W = 128
_NB = 4
_NC = 2
_NS = 16
_LANES = 16


def _sc_hash_gather(table, raw_flat, n_idx):
    mesh = plsc.VectorSubcoreMesh(core_axis_name="core", subcore_axis_name="subcore")
    out_type = jax.ShapeDtypeStruct((n_idx, 128), table.dtype)
    n_tiles = _NC * _NS
    per_tile = n_idx // n_tiles
    n_win = per_tile // _W
    n_outer = n_win // _NB

    @functools.partial(
        pl.kernel,
        out_type=out_type,
        mesh=mesh,
        scratch_types=(
            [
                pltpu.VMEM((_NB, _W), jnp.int32),
                pltpu.VMEM((_NB, _W), jnp.int32),
                pltpu.VMEM((_NB, _W, _EMBED_DIM), jnp.float32),
            ]
            + [pltpu.SemaphoreType.DMA] * (3 * _NB)
        ),
        compiler_params=pltpu.CompilerParams(use_tc_tiling_on_sc=False),
    )
    def k(table_hbm, in_hbm, out_hbm, raw_v, idx_v, rows_v, *sems):
        sem_idx = sems[0:_NB]
        sem_g = sems[_NB : 2 * _NB]
        sem_out = sems[2 * _NB : 3 * _NB]
        wid = lax.axis_index("subcore") * _NC + lax.axis_index("core")
        tbase = wid * per_tile

        def start_idx_dma(w, u):
            pltpu.async_copy(
                in_hbm.at[pl.ds(tbase + w * _W, _W)], raw_v.at[u], sem_idx[u]
            )

        def wait_idx(u):
            pltpu.make_async_copy(
                in_hbm.at[pl.ds(tbase, _W)], raw_v.at[u], sem_idx[u]
            ).wait()

        def start_gather(u):
            pltpu.async_copy(table_hbm.at[idx_v.at[u]], rows_v.at[u], sem_g[u])

        def wait_gather(u):
            pltpu.make_async_copy(
                table_hbm.at[idx_v.at[u]], rows_v.at[u], sem_g[u]
            ).wait()

        def start_out_dma(w, u):
            pltpu.async_copy(
                rows_v.at[u],
                out_hbm.at[pl.ds(tbase + w * _W, _W), pl.ds(0, _EMBED_DIM)],
                sem_out[u],
            )

        def wait_out(u):
            pltpu.make_async_copy(
                rows_v.at[u],
                out_hbm.at[pl.ds(tbase, _W), pl.ds(0, _EMBED_DIM)],
                sem_out[u],
            ).wait()

        def hash_window(u):
            for c in range(_W // _LANES):
                sl = pl.ds(c * _LANES, _LANES)
                v = raw_v[u, sl].astype(jnp.uint32)
                h = (v * jnp.uint32(_HASH_MULT)) % jnp.uint32(_NUM_BINS)
                idx_v[u, sl] = h.astype(jnp.int32)

        for u in range(_NB):
            start_idx_dma(u, u)

        @pl.loop(0, n_outer)
        def _(o):
            for u in range(_NB):
                w = o * _NB + u

                @pl.when(o > 0)
                def _():
                    wait_out(u)

                wait_idx(u)
                hash_window(u)
                start_gather(u)

                up = (u - 1) % _NB
                if u > 0:
                    wait_gather(up)
                    start_out_dma(w - 1, up)
                else:

                    @pl.when(o > 0)
                    def _():
                        wait_gather(up)
                        start_out_dma(w - 1, up)

                @pl.when(o < n_outer - 1)
                def _():
                    start_idx_dma(w + _NB, u)

        last_u = (n_win - 1) % _NB
        wait_gather(last_u)
        start_out_dma(n_win - 1, last_u)
        for u in range(_NB):
            wait_out(u)

    return k(table, raw_flat)


def kernel(inputs, table):
    b, f = inputs.shape
    n = b * f
    out = _sc_hash_gather(table, inputs.reshape(n), n)
    return out[:, :_EMBED_DIM].reshape(b, f, _EMBED_DIM)
